# Initial kernel scaffold; baseline (speedup 1.0000x reference)
import jax
import jax.numpy as jnp
from jax.experimental import pallas as pl


def kernel(hidden_states, W1, b1, ln_g, ln_b, W2, b2, query_origin, memory_embeddings, Wp, bp):
    raise NotImplementedError("write your pallas kernel here")


# speedup vs baseline: 1.9962x; 1.9962x over previous
import functools

import jax
import jax.numpy as jnp
from jax import lax
from jax.experimental import pallas as pl
from jax.experimental.pallas import tpu as pltpu
from jax.experimental.pallas import tpu_sc as plsc

_EPS = 1e-5
_MAX_NORM = 1.0 - 1e-5
_MN2 = _MAX_NORM * _MAX_NORM
_ALPHA = 0.1
_K = 16
_BM = 2048
_RUNW = 128
_PAD_VAL = 100.0


def _front_body(hs_ref, w1_ref, b1_ref, g_ref, bb_ref, w2_ref, b2_ref,
                org_ref, q_ref):
    hs = hs_ref[...]
    pooled = jnp.mean(hs, axis=1)
    h = jnp.dot(pooled, w1_ref[...],
                preferred_element_type=jnp.float32) + b1_ref[...]
    mu = jnp.mean(h, axis=-1, keepdims=True)
    var = jnp.mean((h - mu) ** 2, axis=-1, keepdims=True)
    h = (h - mu) / jnp.sqrt(var + 1e-5) * g_ref[...] + bb_ref[...]
    h = 0.5 * h * (1.0 + lax.erf(h / jnp.sqrt(2.0).astype(jnp.float32)))
    v = jnp.dot(h, w2_ref[...],
                preferred_element_type=jnp.float32) + b2_ref[...]
    vn = jnp.maximum(jnp.sqrt(jnp.sum(v * v, axis=-1, keepdims=True)), _EPS)
    second = jnp.tanh(0.5 * vn) * v / vn
    u = org_ref[...]
    dot_uv = jnp.sum(u * second, axis=-1, keepdims=True)
    nu = jnp.clip(jnp.sum(u * u, axis=-1, keepdims=True), 0.0, _MN2)
    nv = jnp.clip(jnp.sum(second * second, axis=-1, keepdims=True), 0.0, _MN2)
    num = (1.0 + 2.0 * dot_uv + nv) * u + (1.0 - nu) * second
    den = 1.0 + 2.0 * dot_uv + nu * nv
    res = num / jnp.maximum(den, _EPS)
    n = jnp.maximum(jnp.sqrt(jnp.sum(res * res, axis=-1, keepdims=True)), _EPS)
    q_ref[...] = res / jnp.maximum(n / _MAX_NORM, 1.0)


def _topk_body(q_ref, mem_ref, dist_ref, idx_ref, run_v, run_i, *, nm):
    m = pl.program_id(1)

    @pl.when(m == 0)
    def _init():
        run_v[...] = jnp.full(run_v.shape, jnp.inf, jnp.float32)
        run_i[...] = jnp.full(run_i.shape, jnp.iinfo(jnp.int32).max, jnp.int32)

    q = q_ref[...]
    mem = mem_ref[...]
    qq = jnp.sum(q * q, axis=-1, keepdims=True)
    mm = jnp.sum(mem * mem, axis=-1)[None, :]
    qm = lax.dot_general(q, mem, (((1,), (1,)), ((), ())),
                         preferred_element_type=jnp.float32)
    dist_sq = jnp.maximum(qq + mm - 2.0 * qm, 0.0)
    nu = jnp.clip(qq, 0.0, _MN2)
    nv = jnp.clip(mm, 0.0, _MN2)
    den = jnp.maximum((1.0 - nu) * (1.0 - nv), _EPS)
    x = 2.0 * dist_sq / den

    bq = q.shape[0]
    cols = lax.broadcasted_iota(jnp.int32, (bq, _BM), 1) + m * _BM
    sv = jnp.concatenate([run_v[...], x], axis=1)
    si = jnp.concatenate([run_i[...], cols], axis=1)
    big_i = jnp.iinfo(jnp.int32).max
    new_v = []
    new_i = []
    for _ in range(_K):
        vm = jnp.min(sv, axis=1, keepdims=True)
        iw = jnp.where(sv == vm, si, big_i)
        im = jnp.min(iw, axis=1, keepdims=True)
        new_v.append(vm)
        new_i.append(im)
        sv = jnp.where(iw == im, jnp.inf, sv)
    run_v[:, : _K] = jnp.concatenate(new_v, axis=1)
    run_i[:, : _K] = jnp.concatenate(new_i, axis=1)

    @pl.when(m == nm - 1)
    def _fin():
        xc = jnp.maximum(run_v[:, : _K], 1e-12)
        z = 1.0 + xc
        dist = jnp.log(z + jnp.sqrt((z - 1.0) * (z + 1.0)))
        neg = -dist
        e = jnp.exp(neg - jnp.max(neg, axis=1, keepdims=True))
        dist_ref[...] = e / jnp.sum(e, axis=1, keepdims=True)
        idx_ref[...] = run_i[:, : _K]


def _inject_body(hs_ref, r_ref, wp_ref, bp_ref, out_ref):
    mf = jnp.dot(r_ref[...], wp_ref[...],
                 preferred_element_type=jnp.float32) + bp_ref[...]
    out_ref[...] = hs_ref[...] + _ALPHA * mf[:, None, :]


def _sc_retrieve(mem, idx, wts):
    b, k = idx.shape
    d = mem.shape[1]
    mem2 = mem.reshape(mem.shape[0] // 2, 2 * d)
    gidx = jax.lax.shift_right_logical(idx, 1)
    parity = jax.lax.bitwise_and(idx, 1)
    info = plsc.get_sparse_core_info()
    nc, ns = info.num_cores, info.num_subcores
    nw = nc * ns
    qw = b // nw
    rows_per_w = qw * k
    n_chunk = max(1, rows_per_w // 128)
    chunk = rows_per_w // n_chunk
    gidx2 = gidx.reshape(b * k // chunk, chunk)
    pflat = parity.reshape(b * k)
    wflat = wts.reshape(b * k)
    mesh = plsc.VectorSubcoreMesh(core_axis_name="c", subcore_axis_name="s")

    @functools.partial(
        pl.kernel,
        mesh=mesh,
        out_type=jax.ShapeDtypeStruct((b, d), jnp.float32),
        scratch_types=[
            pltpu.VMEM((n_chunk, chunk), jnp.int32),
            pltpu.VMEM((rows_per_w, 2 * d), jnp.float32),
            pltpu.VMEM((rows_per_w,), jnp.float32),
            pltpu.VMEM((rows_per_w,), jnp.int32),
            pltpu.VMEM((qw, d), jnp.float32),
            pltpu.SemaphoreType.DMA,
        ],
    )
    def _sc_k(mem_hbm, idx_hbm, w_hbm, p_hbm, out_hbm, idx_v, rows_v, w_v,
              p_v, acc_v, sem):
        cid = lax.axis_index("c")
        sid = lax.axis_index("s")
        wid = sid * nc + cid
        pltpu.sync_copy(idx_hbm.at[pl.ds(wid * n_chunk, n_chunk)], idx_v)
        pltpu.sync_copy(w_hbm.at[pl.ds(wid * rows_per_w, rows_per_w)], w_v)
        pltpu.sync_copy(p_hbm.at[pl.ds(wid * rows_per_w, rows_per_w)], p_v)
        cps = [
            pltpu.async_copy(mem_hbm.at[idx_v.at[j]],
                             rows_v.at[pl.ds(j * chunk, chunk)], sem)
            for j in range(n_chunk)
        ]
        for cp in cps:
            cp.wait()

        def _one_query(q, _):
            wq = w_v[pl.ds(q * k, k)]
            pq = p_v[pl.ds(q * k, k)] * d
            for j in range(d // 16):
                acc = jnp.zeros((16,), jnp.float32)
                for kk in range(k):
                    acc = acc + wq[kk] * \
                        rows_v[q * k + kk, pl.ds(pq[kk] + j * 16, 16)]
                acc_v[q, pl.ds(j * 16, 16)] = acc
            return _

        lax.fori_loop(0, qw, _one_query, None)
        pltpu.sync_copy(acc_v, out_hbm.at[pl.ds(wid * qw, qw)])

    return _sc_k(mem2, gidx2, wflat, pflat)


def kernel(hidden_states, W1, b1, ln_g, ln_b, W2, b2, query_origin,
           memory_embeddings, Wp, bp):
    b, s, h = hidden_states.shape
    d = W1.shape[1]
    m = memory_embeddings.shape[0]

    bq1 = 128 if b % 128 == 0 else b
    b1r = b1.reshape(1, d)
    gr = ln_g.reshape(1, d)
    br = ln_b.reshape(1, d)
    b2r = b2.reshape(1, d)
    bpr = bp.reshape(1, h)
    query = pl.pallas_call(
        _front_body,
        grid=(b // bq1,),
        in_specs=[
            pl.BlockSpec((bq1, s, h), lambda i: (i, 0, 0)),
            pl.BlockSpec((h, d), lambda i: (0, 0)),
            pl.BlockSpec((1, d), lambda i: (0, 0)),
            pl.BlockSpec((1, d), lambda i: (0, 0)),
            pl.BlockSpec((1, d), lambda i: (0, 0)),
            pl.BlockSpec((d, d), lambda i: (0, 0)),
            pl.BlockSpec((1, d), lambda i: (0, 0)),
            pl.BlockSpec((1, d), lambda i: (0, 0)),
        ],
        out_specs=pl.BlockSpec((bq1, d), lambda i: (i, 0)),
        out_shape=jax.ShapeDtypeStruct((b, d), jnp.float32),
    )(hidden_states, W1, b1r, gr, br, W2, b2r, query_origin)

    mp = ((m + _BM - 1) // _BM) * _BM
    mem_p = memory_embeddings
    if mp != m:
        mem_p = jnp.concatenate(
            [memory_embeddings, jnp.full((mp - m, d), _PAD_VAL, jnp.float32)],
            axis=0)
    nm = mp // _BM
    bq2 = 256 if b % 256 == 0 else b
    nb = b // bq2
    topk_dist, topk_idx = pl.pallas_call(
        functools.partial(_topk_body, nm=nm),
        grid=(nb, nm),
        in_specs=[
            pl.BlockSpec((bq2, d), lambda i, j: (i, 0)),
            pl.BlockSpec((_BM, d), lambda i, j: (j, 0)),
        ],
        out_specs=[
            pl.BlockSpec((bq2, _K), lambda i, j: (i, 0)),
            pl.BlockSpec((bq2, _K), lambda i, j: (i, 0)),
        ],
        out_shape=[
            jax.ShapeDtypeStruct((b, _K), jnp.float32),
            jax.ShapeDtypeStruct((b, _K), jnp.int32),
        ],
        scratch_shapes=[
            pltpu.VMEM((bq2, _RUNW), jnp.float32),
            pltpu.VMEM((bq2, _RUNW), jnp.int32),
        ],
    )(query, mem_p)

    retrieved = _sc_retrieve(mem_p, topk_idx, topk_dist)

    bq4 = 128 if b % 128 == 0 else b
    injected = pl.pallas_call(
        _inject_body,
        grid=(b // bq4,),
        in_specs=[
            pl.BlockSpec((bq4, s, h), lambda i: (i, 0, 0)),
            pl.BlockSpec((bq4, d), lambda i: (i, 0)),
            pl.BlockSpec((d, h), lambda i: (0, 0)),
            pl.BlockSpec((1, h), lambda i: (0, 0)),
        ],
        out_specs=pl.BlockSpec((bq4, s, h), lambda i: (i, 0, 0)),
        out_shape=jax.ShapeDtypeStruct((b, s, h), jnp.float32),
    )(hidden_states, retrieved, Wp, bpr)
    return injected


# speedup vs baseline: 3.9875x; 1.9976x over previous
import functools

import jax
import jax.numpy as jnp
from jax import lax
from jax.experimental import pallas as pl
from jax.experimental.pallas import tpu as pltpu
from jax.experimental.pallas import tpu_sc as plsc

_EPS = 1e-5
_MAX_NORM = 1.0 - 1e-5
_MN2 = _MAX_NORM * _MAX_NORM
_ALPHA = 0.1
_K = 16
_BM = 2048
_PAD_VAL = 100.0


def _front_body(hs_ref, w1_ref, b1_ref, g_ref, bb_ref, w2_ref, b2_ref,
                org_ref, q_ref):
    hs = hs_ref[...]
    pooled = jnp.mean(hs, axis=1)
    h = jnp.dot(pooled, w1_ref[...],
                preferred_element_type=jnp.float32) + b1_ref[...]
    mu = jnp.mean(h, axis=-1, keepdims=True)
    var = jnp.mean((h - mu) ** 2, axis=-1, keepdims=True)
    h = (h - mu) / jnp.sqrt(var + 1e-5) * g_ref[...] + bb_ref[...]
    h = 0.5 * h * (1.0 + lax.erf(h / jnp.sqrt(2.0).astype(jnp.float32)))
    v = jnp.dot(h, w2_ref[...],
                preferred_element_type=jnp.float32) + b2_ref[...]
    vn = jnp.maximum(jnp.sqrt(jnp.sum(v * v, axis=-1, keepdims=True)), _EPS)
    second = jnp.tanh(0.5 * vn) * v / vn
    u = org_ref[...]
    dot_uv = jnp.sum(u * second, axis=-1, keepdims=True)
    nu = jnp.clip(jnp.sum(u * u, axis=-1, keepdims=True), 0.0, _MN2)
    nv = jnp.clip(jnp.sum(second * second, axis=-1, keepdims=True), 0.0, _MN2)
    num = (1.0 + 2.0 * dot_uv + nv) * u + (1.0 - nu) * second
    den = 1.0 + 2.0 * dot_uv + nu * nv
    res = num / jnp.maximum(den, _EPS)
    n = jnp.maximum(jnp.sqrt(jnp.sum(res * res, axis=-1, keepdims=True)), _EPS)
    q_ref[...] = res / jnp.maximum(n / _MAX_NORM, 1.0)


def _blockkeys_body(q_ref, mem_ref, keys_ref):
    q = q_ref[...]
    mem = mem_ref[...]
    qq = jnp.sum(q * q, axis=-1, keepdims=True)
    nu = jnp.clip(qq, 0.0, _MN2)
    mm = jnp.sum(mem * mem, axis=-1)[None, :]
    qm = lax.dot_general(q, mem, (((1,), (1,)), ((), ())),
                         preferred_element_type=jnp.float32)
    dist_sq = jnp.maximum(qq + mm - 2.0 * qm, 0.0)
    nv = jnp.clip(mm, 0.0, _MN2)
    den = jnp.maximum((1.0 - nu) * (1.0 - nv), _EPS)
    x = 2.0 * dist_sq / den

    bq = q.shape[0]
    cols = lax.broadcasted_iota(jnp.int32, (bq, _BM), 1)
    xb = lax.bitcast_convert_type(x, jnp.int32)
    key = lax.bitcast_convert_type(
        lax.bitwise_or(lax.bitwise_and(xb, jnp.int32(-_BM)), cols),
        jnp.float32)
    kvs = []
    for _ in range(_K):
        vm = jnp.min(key, axis=1, keepdims=True)
        kvs.append(vm)
        key = jnp.where(key == vm, jnp.inf, key)
    keys_ref[0, :, :] = jnp.concatenate(kvs, axis=1)


def _gmerge_body(keys_ref, w_ref, idx_ref):
    kf = keys_ref[...]
    lanepos = lax.broadcasted_iota(jnp.int32, kf.shape, 1)
    ki = lax.bitcast_convert_type(kf, jnp.int32)
    gidx = lax.shift_right_logical(lanepos, 4) * _BM + \
        lax.bitwise_and(ki, jnp.int32(_BM - 1))
    big_i = jnp.iinfo(jnp.int32).max
    sv = kf
    si = gidx
    new_v = []
    new_i = []
    for _ in range(_K):
        vm = jnp.min(sv, axis=1, keepdims=True)
        iw = jnp.where(sv == vm, si, big_i)
        im = jnp.min(iw, axis=1, keepdims=True)
        new_v.append(vm)
        new_i.append(im)
        sv = jnp.where(iw == im, jnp.inf, sv)
    kb = lax.bitcast_convert_type(jnp.concatenate(new_v, axis=1), jnp.int32)
    xs = lax.bitcast_convert_type(
        lax.bitwise_and(kb, jnp.int32(-_BM)), jnp.float32)
    xc = jnp.maximum(xs, 1e-12)
    z = 1.0 + xc
    dist = jnp.log(z + jnp.sqrt((z - 1.0) * (z + 1.0)))
    neg = -dist
    e = jnp.exp(neg - jnp.max(neg, axis=1, keepdims=True))
    w_ref[...] = e / jnp.sum(e, axis=1, keepdims=True)
    idx_ref[...] = jnp.concatenate(new_i, axis=1)


def _inject_body(hs_ref, r_ref, wp_ref, bp_ref, out_ref):
    mf = jnp.dot(r_ref[...], wp_ref[...],
                 preferred_element_type=jnp.float32) + bp_ref[...]
    out_ref[...] = hs_ref[...] + _ALPHA * mf[:, None, :]


def _sc_retrieve(mem, idx, wts):
    b, k = idx.shape
    d = mem.shape[1]
    mem2 = mem.reshape(mem.shape[0] // 2, 2 * d)
    gidx = jax.lax.shift_right_logical(idx, 1)
    parity = jax.lax.bitwise_and(idx, 1)
    info = plsc.get_sparse_core_info()
    nc, ns = info.num_cores, info.num_subcores
    nw = nc * ns
    qw = b // nw
    rows_per_w = qw * k
    n_chunk = max(1, rows_per_w // 128)
    chunk = rows_per_w // n_chunk
    gidx2 = gidx.reshape(b * k // chunk, chunk)
    pflat = parity.reshape(b * k)
    wflat = wts.reshape(b * k)
    mesh = plsc.VectorSubcoreMesh(core_axis_name="c", subcore_axis_name="s")

    @functools.partial(
        pl.kernel,
        mesh=mesh,
        out_type=jax.ShapeDtypeStruct((b, d), jnp.float32),
        scratch_types=[
            pltpu.VMEM((n_chunk, chunk), jnp.int32),
            pltpu.VMEM((rows_per_w, 2 * d), jnp.float32),
            pltpu.VMEM((rows_per_w,), jnp.float32),
            pltpu.VMEM((rows_per_w,), jnp.int32),
            pltpu.VMEM((qw, d), jnp.float32),
            pltpu.SemaphoreType.DMA,
        ],
    )
    def _sc_k(mem_hbm, idx_hbm, w_hbm, p_hbm, out_hbm, idx_v, rows_v, w_v,
              p_v, acc_v, sem):
        cid = lax.axis_index("c")
        sid = lax.axis_index("s")
        wid = sid * nc + cid
        pltpu.sync_copy(idx_hbm.at[pl.ds(wid * n_chunk, n_chunk)], idx_v)
        pltpu.sync_copy(w_hbm.at[pl.ds(wid * rows_per_w, rows_per_w)], w_v)
        pltpu.sync_copy(p_hbm.at[pl.ds(wid * rows_per_w, rows_per_w)], p_v)
        cps = [
            pltpu.async_copy(mem_hbm.at[idx_v.at[j]],
                             rows_v.at[pl.ds(j * chunk, chunk)], sem)
            for j in range(n_chunk)
        ]
        for cp in cps:
            cp.wait()

        def _one_query(q, _):
            wq = w_v[pl.ds(q * k, k)]
            pq = p_v[pl.ds(q * k, k)] * d
            for j in range(d // 16):
                acc = jnp.zeros((16,), jnp.float32)
                for kk in range(k):
                    acc = acc + wq[kk] * \
                        rows_v[q * k + kk, pl.ds(pq[kk] + j * 16, 16)]
                acc_v[q, pl.ds(j * 16, 16)] = acc
            return _

        lax.fori_loop(0, qw, _one_query, None)
        pltpu.sync_copy(acc_v, out_hbm.at[pl.ds(wid * qw, qw)])

    return _sc_k(mem2, gidx2, wflat, pflat)


def kernel(hidden_states, W1, b1, ln_g, ln_b, W2, b2, query_origin,
           memory_embeddings, Wp, bp):
    b, s, h = hidden_states.shape
    d = W1.shape[1]
    m = memory_embeddings.shape[0]

    bq1 = 128 if b % 128 == 0 else b
    b1r = b1.reshape(1, d)
    gr = ln_g.reshape(1, d)
    br = ln_b.reshape(1, d)
    b2r = b2.reshape(1, d)
    bpr = bp.reshape(1, h)
    query = pl.pallas_call(
        _front_body,
        grid=(b // bq1,),
        in_specs=[
            pl.BlockSpec((bq1, s, h), lambda i: (i, 0, 0)),
            pl.BlockSpec((h, d), lambda i: (0, 0)),
            pl.BlockSpec((1, d), lambda i: (0, 0)),
            pl.BlockSpec((1, d), lambda i: (0, 0)),
            pl.BlockSpec((1, d), lambda i: (0, 0)),
            pl.BlockSpec((d, d), lambda i: (0, 0)),
            pl.BlockSpec((1, d), lambda i: (0, 0)),
            pl.BlockSpec((1, d), lambda i: (0, 0)),
        ],
        out_specs=pl.BlockSpec((bq1, d), lambda i: (i, 0)),
        out_shape=jax.ShapeDtypeStruct((b, d), jnp.float32),
    )(hidden_states, W1, b1r, gr, br, W2, b2r, query_origin)

    mp = ((m + _BM - 1) // _BM) * _BM
    mem_p = memory_embeddings
    if mp != m:
        mem_p = jnp.concatenate(
            [memory_embeddings, jnp.full((mp - m, d), _PAD_VAL, jnp.float32)],
            axis=0)
    nm = mp // _BM
    bq2 = 256 if b % 256 == 0 else b
    nb = b // bq2
    blk_keys = pl.pallas_call(
        _blockkeys_body,
        grid=(nb, nm),
        in_specs=[
            pl.BlockSpec((bq2, d), lambda i, j: (i, 0)),
            pl.BlockSpec((_BM, d), lambda i, j: (j, 0)),
        ],
        out_specs=pl.BlockSpec((1, bq2, _K), lambda i, j: (j, i, 0)),
        out_shape=jax.ShapeDtypeStruct((nm, b, _K), jnp.float32),
    )(query, mem_p)
    cand_keys = jnp.transpose(blk_keys, (1, 0, 2)).reshape(b, nm * _K)

    topk_dist, topk_idx = pl.pallas_call(
        _gmerge_body,
        grid=(nb,),
        in_specs=[pl.BlockSpec((bq2, nm * _K), lambda i: (i, 0))],
        out_specs=[
            pl.BlockSpec((bq2, _K), lambda i: (i, 0)),
            pl.BlockSpec((bq2, _K), lambda i: (i, 0)),
        ],
        out_shape=[
            jax.ShapeDtypeStruct((b, _K), jnp.float32),
            jax.ShapeDtypeStruct((b, _K), jnp.int32),
        ],
    )(cand_keys)

    retrieved = _sc_retrieve(mem_p, topk_idx, topk_dist)

    bq4 = 128 if b % 128 == 0 else b
    injected = pl.pallas_call(
        _inject_body,
        grid=(b // bq4,),
        in_specs=[
            pl.BlockSpec((bq4, s, h), lambda i: (i, 0, 0)),
            pl.BlockSpec((bq4, d), lambda i: (i, 0)),
            pl.BlockSpec((d, h), lambda i: (0, 0)),
            pl.BlockSpec((1, h), lambda i: (0, 0)),
        ],
        out_specs=pl.BlockSpec((bq4, s, h), lambda i: (i, 0, 0)),
        out_shape=jax.ShapeDtypeStruct((b, s, h), jnp.float32),
    )(hidden_states, retrieved, Wp, bpr)
    return injected


# speedup vs baseline: 6.2251x; 1.5612x over previous
import functools

import jax
import jax.numpy as jnp
from jax import lax
from jax.experimental import pallas as pl
from jax.experimental.pallas import tpu as pltpu
from jax.experimental.pallas import tpu_sc as plsc

_EPS = 1e-5
_MAX_NORM = 1.0 - 1e-5
_MN2 = _MAX_NORM * _MAX_NORM
_ALPHA = 0.1
_K = 16
_BM = 2048
_PAD_VAL = 100.0


def _front_body(hs_ref, w1_ref, b1_ref, g_ref, bb_ref, w2_ref, b2_ref,
                org_ref, q_ref):
    hs = hs_ref[...]
    pooled = jnp.mean(hs, axis=1)
    h = jnp.dot(pooled, w1_ref[...],
                preferred_element_type=jnp.float32) + b1_ref[...]
    mu = jnp.mean(h, axis=-1, keepdims=True)
    var = jnp.mean((h - mu) ** 2, axis=-1, keepdims=True)
    h = (h - mu) / jnp.sqrt(var + 1e-5) * g_ref[...] + bb_ref[...]
    h = 0.5 * h * (1.0 + lax.erf(h / jnp.sqrt(2.0).astype(jnp.float32)))
    v = jnp.dot(h, w2_ref[...],
                preferred_element_type=jnp.float32) + b2_ref[...]
    vn = jnp.maximum(jnp.sqrt(jnp.sum(v * v, axis=-1, keepdims=True)), _EPS)
    second = jnp.tanh(0.5 * vn) * v / vn
    u = org_ref[...]
    dot_uv = jnp.sum(u * second, axis=-1, keepdims=True)
    nu = jnp.clip(jnp.sum(u * u, axis=-1, keepdims=True), 0.0, _MN2)
    nv = jnp.clip(jnp.sum(second * second, axis=-1, keepdims=True), 0.0, _MN2)
    num = (1.0 + 2.0 * dot_uv + nv) * u + (1.0 - nu) * second
    den = 1.0 + 2.0 * dot_uv + nu * nv
    res = num / jnp.maximum(den, _EPS)
    n = jnp.maximum(jnp.sqrt(jnp.sum(res * res, axis=-1, keepdims=True)), _EPS)
    q_ref[...] = res / jnp.maximum(n / _MAX_NORM, 1.0)


def _rclass_body(q_ref, mem_ref, rout_ref, scr, *, nm):
    m = pl.program_id(1)

    @pl.when(m == 0)
    def _init():
        scr[...] = jnp.full(scr.shape, jnp.inf, jnp.float32)

    q = q_ref[...]
    mem = mem_ref[...]
    qq = jnp.sum(q * q, axis=-1, keepdims=True)
    nu = jnp.clip(qq, 0.0, _MN2)
    mm = jnp.sum(mem * mem, axis=-1)[None, :]
    qm = lax.dot_general(q, mem, (((1,), (1,)), ((), ())),
                         preferred_element_type=jnp.float32)
    dist_sq = jnp.maximum(qq + mm - 2.0 * qm, 0.0)
    nv = jnp.clip(mm, 0.0, _MN2)
    den = jnp.maximum((1.0 - nu) * (1.0 - nv), _EPS)
    x = 2.0 * dist_sq / den

    bq = q.shape[0]
    cols = lax.broadcasted_iota(jnp.int32, (bq, _BM), 1)
    xb = lax.bitcast_convert_type(x, jnp.int32)
    low = lax.shift_right_logical(cols, 7) + m * 16
    key = lax.bitcast_convert_type(
        lax.bitwise_or(lax.bitwise_and(xb, jnp.int32(-_BM)), low),
        jnp.float32)
    kr = key.reshape(bq, _BM // 128, 128)
    b1 = jnp.min(kr, axis=1)
    kr2 = jnp.where(kr == b1[:, None, :], jnp.inf, kr)
    b2 = jnp.min(kr2, axis=1)

    r = scr[...]
    r1 = r[:, 0:128]
    r2 = r[:, 128:256]
    r3 = r[:, 256:384]
    r4 = r[:, 384:512]
    for t in (b1, b2):
        n1 = jnp.minimum(r1, t)
        t = jnp.maximum(r1, t)
        n2 = jnp.minimum(r2, t)
        t = jnp.maximum(r2, t)
        n3 = jnp.minimum(r3, t)
        t = jnp.maximum(r3, t)
        n4 = jnp.minimum(r4, t)
        r1, r2, r3, r4 = n1, n2, n3, n4
    out = jnp.concatenate([r1, r2, r3, r4], axis=1)
    scr[...] = out

    @pl.when(m == nm - 1)
    def _fin():
        rout_ref[...] = out


def _gmerge_body(keys_ref, w_ref, idx_ref):
    kf = keys_ref[...]
    lanepos = lax.broadcasted_iota(jnp.int32, kf.shape, 1)
    ki = lax.bitcast_convert_type(kf, jnp.int32)
    lowb = lax.bitwise_and(ki, jnp.int32(_BM - 1))
    gidx = lax.shift_right_logical(lowb, 4) * _BM + \
        lax.bitwise_and(lowb, jnp.int32(15)) * 128 + \
        lax.bitwise_and(lanepos, jnp.int32(127))
    big_i = jnp.iinfo(jnp.int32).max
    sv = kf
    si = gidx
    new_v = []
    new_i = []
    for _ in range(_K):
        vm = jnp.min(sv, axis=1, keepdims=True)
        iw = jnp.where(sv == vm, si, big_i)
        im = jnp.min(iw, axis=1, keepdims=True)
        new_v.append(vm)
        new_i.append(im)
        sv = jnp.where(iw == im, jnp.inf, sv)
    kb = lax.bitcast_convert_type(jnp.concatenate(new_v, axis=1), jnp.int32)
    xs = lax.bitcast_convert_type(
        lax.bitwise_and(kb, jnp.int32(-_BM)), jnp.float32)
    xc = jnp.maximum(xs, 1e-12)
    z = 1.0 + xc
    dist = jnp.log(z + jnp.sqrt((z - 1.0) * (z + 1.0)))
    neg = -dist
    e = jnp.exp(neg - jnp.max(neg, axis=1, keepdims=True))
    w_ref[...] = e / jnp.sum(e, axis=1, keepdims=True)
    idx_ref[...] = jnp.concatenate(new_i, axis=1)


def _inject_body(hs_ref, r_ref, wp_ref, bp_ref, out_ref):
    mf = jnp.dot(r_ref[...], wp_ref[...],
                 preferred_element_type=jnp.float32) + bp_ref[...]
    out_ref[...] = hs_ref[...] + _ALPHA * mf[:, None, :]


def _sc_retrieve(mem, idx, wts):
    b, k = idx.shape
    d = mem.shape[1]
    mem2 = mem.reshape(mem.shape[0] // 2, 2 * d)
    gidx = jax.lax.shift_right_logical(idx, 1)
    parity = jax.lax.bitwise_and(idx, 1)
    info = plsc.get_sparse_core_info()
    nc, ns = info.num_cores, info.num_subcores
    nw = nc * ns
    qw = b // nw
    rows_per_w = qw * k
    n_chunk = max(1, rows_per_w // 128)
    chunk = rows_per_w // n_chunk
    gidx2 = gidx.reshape(b * k // chunk, chunk)
    pflat = parity.reshape(b * k)
    wflat = wts.reshape(b * k)
    mesh = plsc.VectorSubcoreMesh(core_axis_name="c", subcore_axis_name="s")

    @functools.partial(
        pl.kernel,
        mesh=mesh,
        out_type=jax.ShapeDtypeStruct((b, d), jnp.float32),
        scratch_types=[
            pltpu.VMEM((n_chunk, chunk), jnp.int32),
            pltpu.VMEM((rows_per_w, 2 * d), jnp.float32),
            pltpu.VMEM((rows_per_w,), jnp.float32),
            pltpu.VMEM((rows_per_w,), jnp.int32),
            pltpu.VMEM((qw, d), jnp.float32),
            pltpu.SemaphoreType.DMA,
        ],
    )
    def _sc_k(mem_hbm, idx_hbm, w_hbm, p_hbm, out_hbm, idx_v, rows_v, w_v,
              p_v, acc_v, sem):
        cid = lax.axis_index("c")
        sid = lax.axis_index("s")
        wid = sid * nc + cid
        pltpu.sync_copy(idx_hbm.at[pl.ds(wid * n_chunk, n_chunk)], idx_v)
        pltpu.sync_copy(w_hbm.at[pl.ds(wid * rows_per_w, rows_per_w)], w_v)
        pltpu.sync_copy(p_hbm.at[pl.ds(wid * rows_per_w, rows_per_w)], p_v)
        cps = [
            pltpu.async_copy(mem_hbm.at[idx_v.at[j]],
                             rows_v.at[pl.ds(j * chunk, chunk)], sem)
            for j in range(n_chunk)
        ]
        for cp in cps:
            cp.wait()

        def _one_query(q, _):
            wq = w_v[pl.ds(q * k, k)]
            pq = p_v[pl.ds(q * k, k)] * d
            for j in range(d // 16):
                acc = jnp.zeros((16,), jnp.float32)
                for kk in range(k):
                    acc = acc + wq[kk] * \
                        rows_v[q * k + kk, pl.ds(pq[kk] + j * 16, 16)]
                acc_v[q, pl.ds(j * 16, 16)] = acc
            return _

        lax.fori_loop(0, qw, _one_query, None)
        pltpu.sync_copy(acc_v, out_hbm.at[pl.ds(wid * qw, qw)])

    return _sc_k(mem2, gidx2, wflat, pflat)


def kernel(hidden_states, W1, b1, ln_g, ln_b, W2, b2, query_origin,
           memory_embeddings, Wp, bp):
    b, s, h = hidden_states.shape
    d = W1.shape[1]
    m = memory_embeddings.shape[0]

    bq1 = 128 if b % 128 == 0 else b
    b1r = b1.reshape(1, d)
    gr = ln_g.reshape(1, d)
    br = ln_b.reshape(1, d)
    b2r = b2.reshape(1, d)
    bpr = bp.reshape(1, h)
    query = pl.pallas_call(
        _front_body,
        grid=(b // bq1,),
        in_specs=[
            pl.BlockSpec((bq1, s, h), lambda i: (i, 0, 0)),
            pl.BlockSpec((h, d), lambda i: (0, 0)),
            pl.BlockSpec((1, d), lambda i: (0, 0)),
            pl.BlockSpec((1, d), lambda i: (0, 0)),
            pl.BlockSpec((1, d), lambda i: (0, 0)),
            pl.BlockSpec((d, d), lambda i: (0, 0)),
            pl.BlockSpec((1, d), lambda i: (0, 0)),
            pl.BlockSpec((1, d), lambda i: (0, 0)),
        ],
        out_specs=pl.BlockSpec((bq1, d), lambda i: (i, 0)),
        out_shape=jax.ShapeDtypeStruct((b, d), jnp.float32),
    )(hidden_states, W1, b1r, gr, br, W2, b2r, query_origin)

    mp = ((m + _BM - 1) // _BM) * _BM
    mem_p = memory_embeddings
    if mp != m:
        mem_p = jnp.concatenate(
            [memory_embeddings, jnp.full((mp - m, d), _PAD_VAL, jnp.float32)],
            axis=0)
    nm = mp // _BM
    bq2 = 256 if b % 256 == 0 else b
    nb = b // bq2
    rkeys = pl.pallas_call(
        functools.partial(_rclass_body, nm=nm),
        grid=(nb, nm),
        in_specs=[
            pl.BlockSpec((bq2, d), lambda i, j: (i, 0)),
            pl.BlockSpec((_BM, d), lambda i, j: (j, 0)),
        ],
        out_specs=pl.BlockSpec((bq2, 512), lambda i, j: (i, 0)),
        out_shape=jax.ShapeDtypeStruct((b, 512), jnp.float32),
        scratch_shapes=[pltpu.VMEM((bq2, 512), jnp.float32)],
    )(query, mem_p)

    topk_dist, topk_idx = pl.pallas_call(
        _gmerge_body,
        grid=(nb,),
        in_specs=[pl.BlockSpec((bq2, 512), lambda i: (i, 0))],
        out_specs=[
            pl.BlockSpec((bq2, _K), lambda i: (i, 0)),
            pl.BlockSpec((bq2, _K), lambda i: (i, 0)),
        ],
        out_shape=[
            jax.ShapeDtypeStruct((b, _K), jnp.float32),
            jax.ShapeDtypeStruct((b, _K), jnp.int32),
        ],
    )(rkeys)

    retrieved = _sc_retrieve(mem_p, topk_idx, topk_dist)

    bq4 = 128 if b % 128 == 0 else b
    injected = pl.pallas_call(
        _inject_body,
        grid=(b // bq4,),
        in_specs=[
            pl.BlockSpec((bq4, s, h), lambda i: (i, 0, 0)),
            pl.BlockSpec((bq4, d), lambda i: (i, 0)),
            pl.BlockSpec((d, h), lambda i: (0, 0)),
            pl.BlockSpec((1, h), lambda i: (0, 0)),
        ],
        out_specs=pl.BlockSpec((bq4, s, h), lambda i: (i, 0, 0)),
        out_shape=jax.ShapeDtypeStruct((b, s, h), jnp.float32),
    )(hidden_states, retrieved, Wp, bpr)
    return injected


# speedup vs baseline: 6.6750x; 1.0723x over previous
import functools

import jax
import jax.numpy as jnp
from jax import lax
from jax.experimental import pallas as pl
from jax.experimental.pallas import tpu as pltpu
from jax.experimental.pallas import tpu_sc as plsc

_EPS = 1e-5
_MAX_NORM = 1.0 - 1e-5
_MN2 = _MAX_NORM * _MAX_NORM
_ALPHA = 0.1
_K = 16
_BM = 2048


def _front_body(hs_ref, w1_ref, b1_ref, g_ref, bb_ref, w2_ref, b2_ref,
                org_ref, q_ref):
    hs = hs_ref[...]
    pooled = jnp.mean(hs, axis=1)
    h = jnp.dot(pooled, w1_ref[...],
                preferred_element_type=jnp.float32) + b1_ref[...]
    mu = jnp.mean(h, axis=-1, keepdims=True)
    var = jnp.mean((h - mu) ** 2, axis=-1, keepdims=True)
    h = (h - mu) / jnp.sqrt(var + 1e-5) * g_ref[...] + bb_ref[...]
    h = 0.5 * h * (1.0 + lax.erf(h / jnp.sqrt(2.0).astype(jnp.float32)))
    v = jnp.dot(h, w2_ref[...],
                preferred_element_type=jnp.float32) + b2_ref[...]
    vn = jnp.maximum(jnp.sqrt(jnp.sum(v * v, axis=-1, keepdims=True)), _EPS)
    second = jnp.tanh(0.5 * vn) * v / vn
    u = org_ref[...]
    dot_uv = jnp.sum(u * second, axis=-1, keepdims=True)
    nu = jnp.clip(jnp.sum(u * u, axis=-1, keepdims=True), 0.0, _MN2)
    nv = jnp.clip(jnp.sum(second * second, axis=-1, keepdims=True), 0.0, _MN2)
    num = (1.0 + 2.0 * dot_uv + nv) * u + (1.0 - nu) * second
    den = 1.0 + 2.0 * dot_uv + nu * nv
    res = num / jnp.maximum(den, _EPS)
    n = jnp.maximum(jnp.sqrt(jnp.sum(res * res, axis=-1, keepdims=True)), _EPS)
    q_ref[...] = res / jnp.maximum(n / _MAX_NORM, 1.0)


def _rclass_body(q_ref, mem_ref, rout_ref, scr, *, nm, mtot):
    m = pl.program_id(1)

    @pl.when(m == 0)
    def _init():
        scr[...] = jnp.full(scr.shape, jnp.inf, jnp.float32)

    q = q_ref[...]
    mem = mem_ref[...]
    qq = jnp.sum(q * q, axis=-1, keepdims=True)
    nu = jnp.clip(qq, 0.0, _MN2)
    mm = jnp.sum(mem * mem, axis=-1)[None, :]
    qm = lax.dot_general(q, mem, (((1,), (1,)), ((), ())),
                         preferred_element_type=jnp.float32)
    dist_sq = jnp.maximum(qq + mm - 2.0 * qm, 0.0)
    nv = jnp.clip(mm, 0.0, _MN2)
    den = jnp.maximum((1.0 - nu) * (1.0 - nv), _EPS)
    x = 2.0 * dist_sq / den

    bq = q.shape[0]
    cols = lax.broadcasted_iota(jnp.int32, (bq, _BM), 1)
    xb = lax.bitcast_convert_type(x, jnp.int32)
    low = lax.shift_right_logical(cols, 7) + m * 16
    key = lax.bitcast_convert_type(
        lax.bitwise_or(lax.bitwise_and(xb, jnp.int32(-_BM)), low),
        jnp.float32)
    key = jnp.where(cols + m * _BM < mtot, key, jnp.inf)
    kr = key.reshape(bq, _BM // 128, 128)
    b1 = jnp.min(kr, axis=1)
    kr2 = jnp.where(kr == b1[:, None, :], jnp.inf, kr)
    b2 = jnp.min(kr2, axis=1)

    r = scr[...]
    r1 = r[:, 0:128]
    r2 = r[:, 128:256]
    r3 = r[:, 256:384]
    r4 = r[:, 384:512]
    for t in (b1, b2):
        n1 = jnp.minimum(r1, t)
        t = jnp.maximum(r1, t)
        n2 = jnp.minimum(r2, t)
        t = jnp.maximum(r2, t)
        n3 = jnp.minimum(r3, t)
        t = jnp.maximum(r3, t)
        n4 = jnp.minimum(r4, t)
        r1, r2, r3, r4 = n1, n2, n3, n4
    out = jnp.concatenate([r1, r2, r3, r4], axis=1)
    scr[...] = out

    @pl.when(m == nm - 1)
    def _fin():
        rout_ref[...] = out


def _gmerge_body(keys_ref, w_ref, idx_ref):
    kf = keys_ref[...]
    lanepos = lax.broadcasted_iota(jnp.int32, kf.shape, 1)
    ki = lax.bitcast_convert_type(kf, jnp.int32)
    lowb = lax.bitwise_and(ki, jnp.int32(_BM - 1))
    gidx = lax.shift_right_logical(lowb, 4) * _BM + \
        lax.bitwise_and(lowb, jnp.int32(15)) * 128 + \
        lax.bitwise_and(lanepos, jnp.int32(127))
    big_i = jnp.iinfo(jnp.int32).max
    sv = kf
    si = gidx
    new_v = []
    new_i = []
    for _ in range(_K):
        vm = jnp.min(sv, axis=1, keepdims=True)
        iw = jnp.where(sv == vm, si, big_i)
        im = jnp.min(iw, axis=1, keepdims=True)
        new_v.append(vm)
        new_i.append(im)
        sv = jnp.where(iw == im, jnp.inf, sv)
    kb = lax.bitcast_convert_type(jnp.concatenate(new_v, axis=1), jnp.int32)
    xs = lax.bitcast_convert_type(
        lax.bitwise_and(kb, jnp.int32(-_BM)), jnp.float32)
    xc = jnp.maximum(xs, 1e-12)
    z = 1.0 + xc
    dist = jnp.log(z + jnp.sqrt((z - 1.0) * (z + 1.0)))
    neg = -dist
    e = jnp.exp(neg - jnp.max(neg, axis=1, keepdims=True))
    w_ref[...] = e / jnp.sum(e, axis=1, keepdims=True)
    idx_ref[...] = jnp.concatenate(new_i, axis=1)


def _inject_body(hs_ref, r_ref, wp_ref, bp_ref, out_ref):
    mf = jnp.dot(r_ref[...], wp_ref[...],
                 preferred_element_type=jnp.float32) + bp_ref[...]
    out_ref[...] = hs_ref[...] + _ALPHA * mf[:, None, :]


def _sc_retrieve(mem, idx, wts):
    b, k = idx.shape
    d = mem.shape[1]
    mem2 = mem.reshape(mem.shape[0] // 2, 2 * d)
    gidx = jax.lax.shift_right_logical(idx, 1)
    parity = jax.lax.bitwise_and(idx, 1)
    info = plsc.get_sparse_core_info()
    nc, ns = info.num_cores, info.num_subcores
    nw = nc * ns
    qw = b // nw
    rows_per_w = qw * k
    n_chunk = max(1, rows_per_w // 128)
    chunk = rows_per_w // n_chunk
    gidx2 = gidx.reshape(b * k // chunk, chunk)
    pflat = parity.reshape(b * k)
    wflat = wts.reshape(b * k)
    mesh = plsc.VectorSubcoreMesh(core_axis_name="c", subcore_axis_name="s")

    @functools.partial(
        pl.kernel,
        mesh=mesh,
        out_type=jax.ShapeDtypeStruct((b, d), jnp.float32),
        scratch_types=[
            pltpu.VMEM((n_chunk, chunk), jnp.int32),
            pltpu.VMEM((rows_per_w, 2 * d), jnp.float32),
            pltpu.VMEM((rows_per_w,), jnp.float32),
            pltpu.VMEM((rows_per_w,), jnp.int32),
            pltpu.VMEM((qw, d), jnp.float32),
            pltpu.SemaphoreType.DMA,
        ],
    )
    def _sc_k(mem_hbm, idx_hbm, w_hbm, p_hbm, out_hbm, idx_v, rows_v, w_v,
              p_v, acc_v, sem):
        cid = lax.axis_index("c")
        sid = lax.axis_index("s")
        wid = sid * nc + cid
        pltpu.sync_copy(idx_hbm.at[pl.ds(wid * n_chunk, n_chunk)], idx_v)
        pltpu.sync_copy(w_hbm.at[pl.ds(wid * rows_per_w, rows_per_w)], w_v)
        pltpu.sync_copy(p_hbm.at[pl.ds(wid * rows_per_w, rows_per_w)], p_v)
        cps = [
            pltpu.async_copy(mem_hbm.at[idx_v.at[j]],
                             rows_v.at[pl.ds(j * chunk, chunk)], sem)
            for j in range(n_chunk)
        ]
        for cp in cps:
            cp.wait()

        def _one_query(q, _):
            wq = w_v[pl.ds(q * k, k)]
            pq = p_v[pl.ds(q * k, k)] * d
            for j in range(d // 16):
                acc = jnp.zeros((16,), jnp.float32)
                for kk in range(k):
                    acc = acc + wq[kk] * \
                        rows_v[q * k + kk, pl.ds(pq[kk] + j * 16, 16)]
                acc_v[q, pl.ds(j * 16, 16)] = acc
            return _

        lax.fori_loop(0, qw, _one_query, None)
        pltpu.sync_copy(acc_v, out_hbm.at[pl.ds(wid * qw, qw)])

    return _sc_k(mem2, gidx2, wflat, pflat)


def kernel(hidden_states, W1, b1, ln_g, ln_b, W2, b2, query_origin,
           memory_embeddings, Wp, bp):
    b, s, h = hidden_states.shape
    d = W1.shape[1]
    m = memory_embeddings.shape[0]

    bq1 = 128 if b % 128 == 0 else b
    b1r = b1.reshape(1, d)
    gr = ln_g.reshape(1, d)
    br = ln_b.reshape(1, d)
    b2r = b2.reshape(1, d)
    bpr = bp.reshape(1, h)
    query = pl.pallas_call(
        _front_body,
        grid=(b // bq1,),
        in_specs=[
            pl.BlockSpec((bq1, s, h), lambda i: (i, 0, 0)),
            pl.BlockSpec((h, d), lambda i: (0, 0)),
            pl.BlockSpec((1, d), lambda i: (0, 0)),
            pl.BlockSpec((1, d), lambda i: (0, 0)),
            pl.BlockSpec((1, d), lambda i: (0, 0)),
            pl.BlockSpec((d, d), lambda i: (0, 0)),
            pl.BlockSpec((1, d), lambda i: (0, 0)),
            pl.BlockSpec((1, d), lambda i: (0, 0)),
        ],
        out_specs=pl.BlockSpec((bq1, d), lambda i: (i, 0)),
        out_shape=jax.ShapeDtypeStruct((b, d), jnp.float32),
    )(hidden_states, W1, b1r, gr, br, W2, b2r, query_origin)

    nm = (m + _BM - 1) // _BM
    bq2 = 512 if b % 512 == 0 else b
    nb = b // bq2
    rkeys = pl.pallas_call(
        functools.partial(_rclass_body, nm=nm, mtot=m),
        grid=(nb, nm),
        in_specs=[
            pl.BlockSpec((bq2, d), lambda i, j: (i, 0)),
            pl.BlockSpec((_BM, d), lambda i, j: (j, 0)),
        ],
        out_specs=pl.BlockSpec((bq2, 512), lambda i, j: (i, 0)),
        out_shape=jax.ShapeDtypeStruct((b, 512), jnp.float32),
        scratch_shapes=[pltpu.VMEM((bq2, 512), jnp.float32)],
    )(query, memory_embeddings)

    topk_dist, topk_idx = pl.pallas_call(
        _gmerge_body,
        grid=(nb,),
        in_specs=[pl.BlockSpec((bq2, 512), lambda i: (i, 0))],
        out_specs=[
            pl.BlockSpec((bq2, _K), lambda i: (i, 0)),
            pl.BlockSpec((bq2, _K), lambda i: (i, 0)),
        ],
        out_shape=[
            jax.ShapeDtypeStruct((b, _K), jnp.float32),
            jax.ShapeDtypeStruct((b, _K), jnp.int32),
        ],
    )(rkeys)

    retrieved = _sc_retrieve(memory_embeddings, topk_idx, topk_dist)

    bq4 = 128 if b % 128 == 0 else b
    injected = pl.pallas_call(
        _inject_body,
        grid=(b // bq4,),
        in_specs=[
            pl.BlockSpec((bq4, s, h), lambda i: (i, 0, 0)),
            pl.BlockSpec((bq4, d), lambda i: (i, 0)),
            pl.BlockSpec((d, h), lambda i: (0, 0)),
            pl.BlockSpec((1, h), lambda i: (0, 0)),
        ],
        out_specs=pl.BlockSpec((bq4, s, h), lambda i: (i, 0, 0)),
        out_shape=jax.ShapeDtypeStruct((b, s, h), jnp.float32),
    )(hidden_states, retrieved, Wp, bpr)
    return injected


# speedup vs baseline: 8.4814x; 1.2706x over previous
import functools

import jax
import jax.numpy as jnp
from jax import lax
from jax.experimental import pallas as pl
from jax.experimental.pallas import tpu as pltpu
from jax.experimental.pallas import tpu_sc as plsc

_EPS = 1e-5
_MAX_NORM = 1.0 - 1e-5
_MN2 = _MAX_NORM * _MAX_NORM
_ALPHA = 0.1
_K = 16
_BM = 2048


def _front_body(hs_ref, w1_ref, b1_ref, g_ref, bb_ref, w2_ref, b2_ref,
                org_ref, q_ref):
    hs = hs_ref[...]
    pooled = jnp.mean(hs, axis=1)
    h = jnp.dot(pooled, w1_ref[...],
                preferred_element_type=jnp.float32) + b1_ref[...]
    mu = jnp.mean(h, axis=-1, keepdims=True)
    var = jnp.mean((h - mu) ** 2, axis=-1, keepdims=True)
    h = (h - mu) / jnp.sqrt(var + 1e-5) * g_ref[...] + bb_ref[...]
    h = 0.5 * h * (1.0 + lax.erf(h / jnp.sqrt(2.0).astype(jnp.float32)))
    v = jnp.dot(h, w2_ref[...],
                preferred_element_type=jnp.float32) + b2_ref[...]
    vn = jnp.maximum(jnp.sqrt(jnp.sum(v * v, axis=-1, keepdims=True)), _EPS)
    second = jnp.tanh(0.5 * vn) * v / vn
    u = org_ref[...]
    dot_uv = jnp.sum(u * second, axis=-1, keepdims=True)
    nu = jnp.clip(jnp.sum(u * u, axis=-1, keepdims=True), 0.0, _MN2)
    nv = jnp.clip(jnp.sum(second * second, axis=-1, keepdims=True), 0.0, _MN2)
    num = (1.0 + 2.0 * dot_uv + nv) * u + (1.0 - nu) * second
    den = 1.0 + 2.0 * dot_uv + nu * nv
    res = num / jnp.maximum(den, _EPS)
    n = jnp.maximum(jnp.sqrt(jnp.sum(res * res, axis=-1, keepdims=True)), _EPS)
    q_ref[...] = res / jnp.maximum(n / _MAX_NORM, 1.0)


def _rclass_body(q_ref, mem_ref, rout_ref, scr, *, nm, mtot):
    m = pl.program_id(1)

    @pl.when(m == 0)
    def _init():
        scr[...] = jnp.full(scr.shape, jnp.inf, jnp.float32)

    q = q_ref[...]
    mem = mem_ref[...]
    qq = jnp.sum(q * q, axis=-1, keepdims=True)
    nu = jnp.clip(qq, 0.0, _MN2)
    mm = jnp.sum(mem * mem, axis=-1)[None, :]
    qm = lax.dot_general(q, mem, (((1,), (1,)), ((), ())),
                         preferred_element_type=jnp.float32)
    dist_sq = jnp.maximum(qq + mm - 2.0 * qm, 0.0)
    nv = jnp.clip(mm, 0.0, _MN2)
    den = jnp.maximum((1.0 - nu) * (1.0 - nv), _EPS)
    x = 2.0 * dist_sq / den

    bq = q.shape[0]
    cols = lax.broadcasted_iota(jnp.int32, (bq, _BM), 1)
    xb = lax.bitcast_convert_type(x, jnp.int32)
    low = lax.shift_right_logical(cols, 7) + m * 16
    key = lax.bitcast_convert_type(
        lax.bitwise_or(lax.bitwise_and(xb, jnp.int32(-_BM)), low),
        jnp.float32)
    key = jnp.where(cols + m * _BM < mtot, key, jnp.inf)
    ng = _BM // 128
    b1 = key[:, 0:128]
    for s in range(1, ng):
        b1 = jnp.minimum(b1, key[:, 128 * s:128 * (s + 1)])
    b1t = jnp.concatenate([b1] * ng, axis=1)
    key2 = jnp.where(key == b1t, jnp.inf, key)
    b2 = key2[:, 0:128]
    for s in range(1, ng):
        b2 = jnp.minimum(b2, key2[:, 128 * s:128 * (s + 1)])

    r = scr[...]
    r1 = r[:, 0:128]
    r2 = r[:, 128:256]
    r3 = r[:, 256:384]
    r4 = r[:, 384:512]
    for t in (b1, b2):
        n1 = jnp.minimum(r1, t)
        t = jnp.maximum(r1, t)
        n2 = jnp.minimum(r2, t)
        t = jnp.maximum(r2, t)
        n3 = jnp.minimum(r3, t)
        t = jnp.maximum(r3, t)
        n4 = jnp.minimum(r4, t)
        r1, r2, r3, r4 = n1, n2, n3, n4
    out = jnp.concatenate([r1, r2, r3, r4], axis=1)
    scr[...] = out

    @pl.when(m == nm - 1)
    def _fin():
        rout_ref[...] = out


def _gmerge_body(keys_ref, w_ref, idx_ref):
    kf = keys_ref[...]
    lanepos = lax.broadcasted_iota(jnp.int32, kf.shape, 1)
    ki = lax.bitcast_convert_type(kf, jnp.int32)
    lowb = lax.bitwise_and(ki, jnp.int32(_BM - 1))
    gidx = lax.shift_right_logical(lowb, 4) * _BM + \
        lax.bitwise_and(lowb, jnp.int32(15)) * 128 + \
        lax.bitwise_and(lanepos, jnp.int32(127))
    big_i = jnp.iinfo(jnp.int32).max
    sv = kf
    si = gidx
    new_v = []
    new_i = []
    for _ in range(_K):
        vm = jnp.min(sv, axis=1, keepdims=True)
        iw = jnp.where(sv == vm, si, big_i)
        im = jnp.min(iw, axis=1, keepdims=True)
        new_v.append(vm)
        new_i.append(im)
        sv = jnp.where(iw == im, jnp.inf, sv)
    kb = lax.bitcast_convert_type(jnp.concatenate(new_v, axis=1), jnp.int32)
    xs = lax.bitcast_convert_type(
        lax.bitwise_and(kb, jnp.int32(-_BM)), jnp.float32)
    xc = jnp.maximum(xs, 1e-12)
    z = 1.0 + xc
    dist = jnp.log(z + jnp.sqrt((z - 1.0) * (z + 1.0)))
    neg = -dist
    e = jnp.exp(neg - jnp.max(neg, axis=1, keepdims=True))
    w_ref[...] = e / jnp.sum(e, axis=1, keepdims=True)
    idx_ref[...] = jnp.concatenate(new_i, axis=1)


def _inject_body(hs_ref, r_ref, wp_ref, bp_ref, out_ref):
    mf = jnp.dot(r_ref[...], wp_ref[...],
                 preferred_element_type=jnp.float32) + bp_ref[...]
    out_ref[...] = hs_ref[...] + _ALPHA * mf[:, None, :]


def _sc_retrieve(mem, idx, wts):
    b, k = idx.shape
    d = mem.shape[1]
    mem2 = mem.reshape(mem.shape[0] // 2, 2 * d)
    gidx = jax.lax.shift_right_logical(idx, 1)
    parity = jax.lax.bitwise_and(idx, 1)
    info = plsc.get_sparse_core_info()
    nc, ns = info.num_cores, info.num_subcores
    nw = nc * ns
    qw = b // nw
    rows_per_w = qw * k
    n_chunk = max(1, rows_per_w // 128)
    chunk = rows_per_w // n_chunk
    gidx2 = gidx.reshape(b * k // chunk, chunk)
    pflat = parity.reshape(b * k)
    wflat = wts.reshape(b * k)
    mesh = plsc.VectorSubcoreMesh(core_axis_name="c", subcore_axis_name="s")

    @functools.partial(
        pl.kernel,
        mesh=mesh,
        out_type=jax.ShapeDtypeStruct((b, d), jnp.float32),
        scratch_types=[
            pltpu.VMEM((n_chunk, chunk), jnp.int32),
            pltpu.VMEM((rows_per_w, 2 * d), jnp.float32),
            pltpu.VMEM((rows_per_w,), jnp.float32),
            pltpu.VMEM((rows_per_w,), jnp.int32),
            pltpu.VMEM((qw, d), jnp.float32),
            pltpu.SemaphoreType.DMA,
        ],
    )
    def _sc_k(mem_hbm, idx_hbm, w_hbm, p_hbm, out_hbm, idx_v, rows_v, w_v,
              p_v, acc_v, sem):
        cid = lax.axis_index("c")
        sid = lax.axis_index("s")
        wid = sid * nc + cid
        pltpu.sync_copy(idx_hbm.at[pl.ds(wid * n_chunk, n_chunk)], idx_v)
        pltpu.sync_copy(w_hbm.at[pl.ds(wid * rows_per_w, rows_per_w)], w_v)
        pltpu.sync_copy(p_hbm.at[pl.ds(wid * rows_per_w, rows_per_w)], p_v)
        cps = [
            pltpu.async_copy(mem_hbm.at[idx_v.at[j]],
                             rows_v.at[pl.ds(j * chunk, chunk)], sem)
            for j in range(n_chunk)
        ]
        for cp in cps:
            cp.wait()

        def _one_query(q, _):
            wq = w_v[pl.ds(q * k, k)]
            pq = p_v[pl.ds(q * k, k)] * d
            for j in range(d // 16):
                acc = jnp.zeros((16,), jnp.float32)
                for kk in range(k):
                    acc = acc + wq[kk] * \
                        rows_v[q * k + kk, pl.ds(pq[kk] + j * 16, 16)]
                acc_v[q, pl.ds(j * 16, 16)] = acc
            return _

        lax.fori_loop(0, qw, _one_query, None)
        pltpu.sync_copy(acc_v, out_hbm.at[pl.ds(wid * qw, qw)])

    return _sc_k(mem2, gidx2, wflat, pflat)


def kernel(hidden_states, W1, b1, ln_g, ln_b, W2, b2, query_origin,
           memory_embeddings, Wp, bp):
    b, s, h = hidden_states.shape
    d = W1.shape[1]
    m = memory_embeddings.shape[0]

    bq1 = 128 if b % 128 == 0 else b
    b1r = b1.reshape(1, d)
    gr = ln_g.reshape(1, d)
    br = ln_b.reshape(1, d)
    b2r = b2.reshape(1, d)
    bpr = bp.reshape(1, h)
    query = pl.pallas_call(
        _front_body,
        grid=(b // bq1,),
        in_specs=[
            pl.BlockSpec((bq1, s, h), lambda i: (i, 0, 0)),
            pl.BlockSpec((h, d), lambda i: (0, 0)),
            pl.BlockSpec((1, d), lambda i: (0, 0)),
            pl.BlockSpec((1, d), lambda i: (0, 0)),
            pl.BlockSpec((1, d), lambda i: (0, 0)),
            pl.BlockSpec((d, d), lambda i: (0, 0)),
            pl.BlockSpec((1, d), lambda i: (0, 0)),
            pl.BlockSpec((1, d), lambda i: (0, 0)),
        ],
        out_specs=pl.BlockSpec((bq1, d), lambda i: (i, 0)),
        out_shape=jax.ShapeDtypeStruct((b, d), jnp.float32),
    )(hidden_states, W1, b1r, gr, br, W2, b2r, query_origin)

    nm = (m + _BM - 1) // _BM
    bq2 = 512 if b % 512 == 0 else b
    nb = b // bq2
    rkeys = pl.pallas_call(
        functools.partial(_rclass_body, nm=nm, mtot=m),
        grid=(nb, nm),
        in_specs=[
            pl.BlockSpec((bq2, d), lambda i, j: (i, 0)),
            pl.BlockSpec((_BM, d), lambda i, j: (j, 0)),
        ],
        out_specs=pl.BlockSpec((bq2, 512), lambda i, j: (i, 0)),
        out_shape=jax.ShapeDtypeStruct((b, 512), jnp.float32),
        scratch_shapes=[pltpu.VMEM((bq2, 512), jnp.float32)],
    )(query, memory_embeddings)

    topk_dist, topk_idx = pl.pallas_call(
        _gmerge_body,
        grid=(nb,),
        in_specs=[pl.BlockSpec((bq2, 512), lambda i: (i, 0))],
        out_specs=[
            pl.BlockSpec((bq2, _K), lambda i: (i, 0)),
            pl.BlockSpec((bq2, _K), lambda i: (i, 0)),
        ],
        out_shape=[
            jax.ShapeDtypeStruct((b, _K), jnp.float32),
            jax.ShapeDtypeStruct((b, _K), jnp.int32),
        ],
    )(rkeys)

    retrieved = _sc_retrieve(memory_embeddings, topk_idx, topk_dist)

    bq4 = 128 if b % 128 == 0 else b
    injected = pl.pallas_call(
        _inject_body,
        grid=(b // bq4,),
        in_specs=[
            pl.BlockSpec((bq4, s, h), lambda i: (i, 0, 0)),
            pl.BlockSpec((bq4, d), lambda i: (i, 0)),
            pl.BlockSpec((d, h), lambda i: (0, 0)),
            pl.BlockSpec((1, h), lambda i: (0, 0)),
        ],
        out_specs=pl.BlockSpec((bq4, s, h), lambda i: (i, 0, 0)),
        out_shape=jax.ShapeDtypeStruct((b, s, h), jnp.float32),
    )(hidden_states, retrieved, Wp, bpr)
    return injected
